# baseline (speedup 1.0000x reference)
import functools

import jax
import jax.numpy as jnp
from jax import lax
from jax.experimental import pallas as pl
from jax.experimental.pallas import tpu as pltpu
from jax.experimental.pallas import tpu_sc as plsc


def _seq_embed_sc(idx2, token_table, pos_table, *, n_rows, depth, length):
    info = plsc.get_sparse_core_info()
    num_cores = info.num_cores
    num_workers = info.num_cores * info.num_subcores
    per_w = n_rows // num_workers
    chunk = 4 * length
    gsize = length // 2
    ngather = chunk // gsize
    n_chunks = per_w // chunk
    nseq = chunk // length

    mesh = plsc.VectorSubcoreMesh(core_axis_name="c", subcore_axis_name="s")

    @functools.partial(
        pl.kernel,
        out_type=jax.ShapeDtypeStruct((n_rows, depth), jnp.float32),
        mesh=mesh,
        scratch_types=[
            pltpu.VMEM((ngather, gsize), jnp.int32),
            pltpu.VMEM((chunk, depth), jnp.float32),
            pltpu.VMEM((length, depth), jnp.float32),
            pltpu.SemaphoreType.DMA,
        ],
        compiler_params=pltpu.CompilerParams(use_tc_tiling_on_sc=False),
    )
    def body(idx_hbm, tok_hbm, pos_hbm, out_hbm, idx_v, rows_v, pos_v, sem):
        wid = lax.axis_index("s") * num_cores + lax.axis_index("c")
        w0 = wid * per_w
        pltpu.sync_copy(pos_hbm, pos_v)

        def chunk_body(ci, carry):
            base = pl.multiple_of(w0 + ci * chunk, chunk)
            irow = pl.multiple_of(base // gsize, ngather)
            pltpu.sync_copy(idx_hbm.at[pl.ds(irow, ngather)], idx_v)
            copies = []
            for g in range(ngather):
                copies.append(
                    pltpu.async_copy(
                        tok_hbm.at[idx_v.at[g]],
                        rows_v.at[pl.ds(g * gsize, gsize)],
                        sem,
                    )
                )
            for cp in copies:
                cp.wait()

            def add_body(r, c):
                for q in range(depth // 16):
                    sl = pl.ds(q * 16, 16)
                    p = pos_v[r, sl]
                    for s in range(nseq):
                        rows_v[s * length + r, sl] += p
                return c

            lax.fori_loop(0, length, add_body, 0, unroll=2)
            pltpu.sync_copy(rows_v, out_hbm.at[pl.ds(base, chunk)])
            return carry

        lax.fori_loop(0, n_chunks, chunk_body, 0)

    return body(idx2, token_table, pos_table)


def kernel(seq, token_table, pos_table):
    b, length = seq.shape
    _, depth = token_table.shape
    n_rows = b * length
    gsize = length // 2
    idx2 = seq.reshape(n_rows // gsize, gsize)
    out = _seq_embed_sc(
        idx2, token_table, pos_table,
        n_rows=n_rows, depth=depth, length=length,
    )
    return out.reshape(b, length, depth)


# speedup vs baseline: 1.0606x; 1.0606x over previous
import functools

import jax
import jax.numpy as jnp
from jax import lax
from jax.experimental import pallas as pl
from jax.experimental.pallas import tpu as pltpu
from jax.experimental.pallas import tpu_sc as plsc


def kernel(seq, token_table, pos_table):
    b, length = seq.shape
    _, depth = token_table.shape

    info = plsc.get_sparse_core_info()
    num_cores = info.num_cores
    num_workers = info.num_cores * info.num_subcores
    seqs_per_w = b // num_workers
    nseq = 4
    chunk = nseq * length
    n_chunks = seqs_per_w // nseq
    g0 = 104
    pieces = tuple((s, off, ln)
                   for s in range(nseq)
                   for off, ln in ((0, g0), (g0, length - g0)))

    mesh = plsc.VectorSubcoreMesh(core_axis_name="c", subcore_axis_name="s")

    @functools.partial(
        pl.kernel,
        out_type=jax.ShapeDtypeStruct((b, length, depth), jnp.float32),
        mesh=mesh,
        scratch_types=[
            pltpu.VMEM((nseq, length), jnp.int32),
            pltpu.VMEM((nseq, length), jnp.int32),
            pltpu.VMEM((chunk, depth), jnp.float32),
            pltpu.VMEM((chunk, depth), jnp.float32),
            pltpu.VMEM((length, depth), jnp.float32),
            pltpu.SemaphoreType.DMA,
            pltpu.SemaphoreType.DMA,
        ],
        compiler_params=pltpu.CompilerParams(use_tc_tiling_on_sc=False),
    )
    def body(seq_hbm, tok_hbm, pos_hbm, out_hbm,
             idx0, idx1, rows0, rows1, pos_v, sem0, sem1):
        wid = lax.axis_index("s") * num_cores + lax.axis_index("c")
        w0 = wid * seqs_per_w
        pltpu.sync_copy(pos_hbm, pos_v)

        def fire(c, idx_v, rows_v, sem):
            b0 = pl.multiple_of(w0 + c * nseq, nseq)
            pltpu.sync_copy(seq_hbm.at[pl.ds(b0, nseq)], idx_v)
            for s, off, ln in pieces:
                pltpu.async_copy(
                    tok_hbm.at[idx_v.at[s, pl.ds(off, ln)]],
                    rows_v.at[pl.ds(s * length + off, ln)],
                    sem,
                )

        def drain(rows_v, sem):
            for s, off, ln in pieces:
                pltpu.make_async_copy(
                    tok_hbm.at[pl.ds(0, ln)],
                    rows_v.at[pl.ds(s * length + off, ln)],
                    sem,
                ).wait()

        def process(c, rows_v):
            b0 = pl.multiple_of(w0 + c * nseq, nseq)

            def add_body(r, carry):
                for q in range(depth // 16):
                    sl = pl.ds(q * 16, 16)
                    p = pos_v[r, sl]
                    for s in range(nseq):
                        rows_v[s * length + r, sl] += p
                return carry

            lax.fori_loop(0, length, add_body, 0, unroll=2)
            for s in range(nseq):
                pltpu.sync_copy(
                    rows_v.at[pl.ds(s * length, length)],
                    out_hbm.at[b0 + s],
                )

        fire(0, idx0, rows0, sem0)

        def chunk_body(k, carry):
            @pl.when(k % 2 == 0)
            def _():
                @pl.when(k + 1 < n_chunks)
                def _():
                    fire(k + 1, idx1, rows1, sem1)
                drain(rows0, sem0)
                process(k, rows0)

            @pl.when(k % 2 == 1)
            def _():
                @pl.when(k + 1 < n_chunks)
                def _():
                    fire(k + 1, idx0, rows0, sem0)
                drain(rows1, sem1)
                process(k, rows1)

            return carry

        lax.fori_loop(0, n_chunks, chunk_body, 0)

    return body(seq, token_table, pos_table)


# speedup vs baseline: 1.0650x; 1.0041x over previous
import functools

import jax
import jax.numpy as jnp
from jax import lax
from jax.experimental import pallas as pl
from jax.experimental.pallas import tpu as pltpu
from jax.experimental.pallas import tpu_sc as plsc


def kernel(seq, token_table, pos_table):
    b, length = seq.shape
    _, depth = token_table.shape

    info = plsc.get_sparse_core_info()
    num_cores = info.num_cores
    num_workers = info.num_cores * info.num_subcores
    seqs_per_w = b // num_workers
    nseq = 4
    chunk = nseq * length
    n_chunks = seqs_per_w // nseq
    g0 = 104
    pieces = tuple((s, off, ln)
                   for s in range(nseq)
                   for off, ln in ((0, g0), (g0, length - g0)))

    mesh = plsc.VectorSubcoreMesh(core_axis_name="c", subcore_axis_name="s")

    @functools.partial(
        pl.kernel,
        out_type=jax.ShapeDtypeStruct((b, length, depth), jnp.float32),
        mesh=mesh,
        scratch_types=[
            pltpu.VMEM((nseq, length), jnp.int32),
            pltpu.VMEM((nseq, length), jnp.int32),
            pltpu.VMEM((chunk, depth), jnp.float32),
            pltpu.VMEM((chunk, depth), jnp.float32),
            pltpu.VMEM((length, depth), jnp.float32),
            pltpu.SemaphoreType.DMA,
            pltpu.SemaphoreType.DMA,
            pltpu.SemaphoreType.DMA,
            pltpu.SemaphoreType.DMA,
        ],
        compiler_params=pltpu.CompilerParams(use_tc_tiling_on_sc=False),
    )
    def body(seq_hbm, tok_hbm, pos_hbm, out_hbm,
             idx0, idx1, rows0, rows1, pos_v, sem0, sem1, osem0, osem1):
        wid = lax.axis_index("s") * num_cores + lax.axis_index("c")
        w0 = wid * seqs_per_w
        pltpu.sync_copy(pos_hbm, pos_v)

        def fire(c, idx_v, rows_v, sem):
            b0 = pl.multiple_of(w0 + c * nseq, nseq)
            pltpu.sync_copy(seq_hbm.at[pl.ds(b0, nseq)], idx_v)
            for s, off, ln in pieces:
                pltpu.async_copy(
                    tok_hbm.at[idx_v.at[s, pl.ds(off, ln)]],
                    rows_v.at[pl.ds(s * length + off, ln)],
                    sem,
                )

        def drain(rows_v, sem):
            for s, off, ln in pieces:
                pltpu.make_async_copy(
                    tok_hbm.at[pl.ds(0, ln)],
                    rows_v.at[pl.ds(s * length + off, ln)],
                    sem,
                ).wait()

        def process(c, rows_v, osem):
            b0 = pl.multiple_of(w0 + c * nseq, nseq)

            def add_body(r, carry):
                for q in range(depth // 16):
                    sl = pl.ds(q * 16, 16)
                    p = pos_v[r, sl]
                    for s in range(nseq):
                        rows_v[s * length + r, sl] += p
                return carry

            lax.fori_loop(0, length, add_body, 0, unroll=2)
            for s in range(nseq):
                pltpu.async_copy(
                    rows_v.at[pl.ds(s * length, length)],
                    out_hbm.at[b0 + s],
                    osem,
                )

        def owait(rows_v, osem):
            for s in range(nseq):
                pltpu.make_async_copy(
                    rows_v.at[pl.ds(s * length, length)],
                    out_hbm.at[0],
                    osem,
                ).wait()

        fire(0, idx0, rows0, sem0)

        def chunk_body(k, carry):
            @pl.when(k % 2 == 0)
            def _():
                @pl.when((k + 1 < n_chunks) & (k >= 1))
                def _():
                    owait(rows1, osem1)

                @pl.when(k + 1 < n_chunks)
                def _():
                    fire(k + 1, idx1, rows1, sem1)
                drain(rows0, sem0)
                process(k, rows0, osem0)

            @pl.when(k % 2 == 1)
            def _():
                @pl.when(k + 1 < n_chunks)
                def _():
                    owait(rows0, osem0)
                    fire(k + 1, idx0, rows0, sem0)
                drain(rows1, sem1)
                process(k, rows1, osem1)

            return carry

        lax.fori_loop(0, n_chunks, chunk_body, 0)
        owait(rows0, osem0)
        owait(rows1, osem1)

    return body(seq, token_table, pos_table)
